# baseline (speedup 1.0000x reference)
import functools

import jax
import jax.numpy as jnp
from jax import lax
from jax.experimental import pallas as pl
from jax.experimental.pallas import tpu as pltpu
from jax.experimental.pallas import tpu_sc as plsc

N = 10000
D = 128
W = D + 16
GW = 2 * D
NXV = 10016
NP = 10016
CH = 32
NW = 32
RZ = NP // 16


def _z(i):
    return i * 0


def _prep_body(x_ref, w_ref, u_ref, gxv_ref, xv_ref):
    xw = jnp.dot(x_ref[...], w_ref[...],
                 preferred_element_type=jnp.float32,
                 precision=lax.Precision.HIGHEST)
    gxv_ref[:, :D] = xw[:, :D] - xw[:, D:]
    gxv_ref[:, D:2 * D] = xw[:, D:]
    uv = u_ref[:, 0:1] - u_ref[:, 1:2]
    xv = jnp.dot(x_ref[...], uv,
                 preferred_element_type=jnp.float32,
                 precision=lax.Precision.HIGHEST)
    xv_ref[...] = xv


def _prep(x, weight, u):
    R = 1000
    return pl.pallas_call(
        _prep_body,
        grid=(N // R,),
        in_specs=[
            pl.BlockSpec((R, D), lambda i: (i, _z(i))),
            pl.BlockSpec((D, 2 * D), lambda i: (_z(i), _z(i))),
            pl.BlockSpec((D, 2), lambda i: (_z(i), _z(i))),
        ],
        out_specs=[
            pl.BlockSpec((R, GW), lambda i: (i, _z(i))),
            pl.BlockSpec((R, 1), lambda i: (i, _z(i))),
        ],
        out_shape=[
            jax.ShapeDtypeStruct((N, GW), jnp.float32),
            jax.ShapeDtypeStruct((N, 1), jnp.float32),
        ],
    )(x, weight, u)


def _sc_body(nch, gxv_hbm, xv_hbm, eidx_hbm, cd_hbm, z_hbm,
             acc_out,
             ib, dstm0, dstm1, rows0, rows1,
             msg0, msg1, xvl, cd_v, acc_sh,
             semi, semg0, semg1, sems0, sems1):
    i32 = jnp.int32
    c_id = lax.axis_index("c")
    s_id = lax.axis_index("s")
    wid = c_id * i32(16) + s_id

    dstm = (dstm0, dstm1)
    rows = (rows0, rows1)
    msg = (msg0, msg1)
    semg = (semg0, semg1)
    sems = (sems0, sems1)

    zb = s_id * i32(RZ)
    pltpu.sync_copy(z_hbm.at[pl.ds(zb, RZ)], acc_sh.at[pl.ds(zb, RZ)])
    pltpu.sync_copy(xv_hbm, xvl)
    pltpu.sync_copy(cd_hbm, cd_v)
    ones16 = jnp.ones((16,), jnp.float32)
    for p in (0, 1):
        for r in range(CH):
            msg[p][i32(r), pl.ds(D, 16)] = ones16
    plsc.subcore_barrier()

    nbat = nch // 8
    bwords = 8 * 2 * CH
    bbase = wid * i32(nbat)

    def ibatch_copy(k):
        return pltpu.make_async_copy(
            eidx_hbm.at[pl.ds((bbase + k) * i32(bwords), bwords)],
            ib.at[lax.rem(k, i32(3))], semi)

    def _slot_off(t):
        return lax.rem(lax.shift_right_logical(t, i32(3)), i32(3)), \
            lax.mul(lax.bitwise_and(t, i32(7)), i32(2 * CH))

    def rows_copy(t, p):
        bq, off = _slot_off(t)
        return pltpu.make_async_copy(
            gxv_hbm.at[ib.at[bq, pl.ds(off, CH)]], rows[p], semg[p])

    def scat_start(p):
        pltpu.async_copy(msg[p], acc_sh.at[dstm[p]], sems[p], add=True)

    def scat_wait(p):
        pltpu.make_async_copy(msg[p], acc_sh.at[dstm[p]], sems[p]).wait()

    def start_gathers(t, p):
        rows_copy(t, p).start()

    def wait_gathers(p):
        rows_copy(i32(0), p).wait()

    def compute(t, p):
        cdv = cd_v[...]
        bq, off = _slot_off(t)
        for g in range(CH // 16):
            srcv = ib[bq, pl.ds(off + i32(g * 16), 16)]
            dstv = ib[bq, pl.ds(off + i32(CH + g * 16), 16)]
            xvs = plsc.load_gather(xvl, [srcv])
            xvdv = plsc.load_gather(xvl, [dstv])
            q = 1.0 / (1.0 + jnp.exp(-(xvs - xvdv + cdv)))
            dstm[p][pl.ds(g * 16, 16)] = jnp.where(srcv != dstv, dstv, i32(N))
            for e in range(16):
                qe = q[e]
                r = g * 16 + e
                for k in range(D // 16):
                    col = k * 16
                    gseg = rows[p][r, pl.ds(col, 16)]
                    bseg = rows[p][r, pl.ds(D + col, 16)]
                    msg[p][r, pl.ds(col, 16)] = bseg + qe * gseg

    ibatch_copy(i32(0)).start()
    ibatch_copy(i32(1)).start()
    ibatch_copy(i32(0)).wait()
    start_gathers(i32(0), 0)

    def pair(i2, carry):
        a = i2 * i32(2)

        @pl.when(lax.rem(i2, i32(4)) == i32(0))
        def _():
            b = lax.div(i2, i32(4))
            ibatch_copy(i32(0)).wait()
            ibatch_copy(b + i32(2)).start()

        wait_gathers(0)

        @pl.when(a >= i32(2))
        def _():
            scat_wait(0)

        start_gathers(a + i32(1), 1)
        compute(a, 0)
        scat_start(0)

        wait_gathers(1)

        @pl.when(a >= i32(1))
        def _():
            scat_wait(1)

        start_gathers(a + i32(2), 0)
        compute(a + i32(1), 1)
        scat_start(1)
        return carry

    lax.fori_loop(jnp.int32(0), jnp.int32(nch // 2), pair, 0)

    ibatch_copy(i32(0)).wait()
    wait_gathers(0)
    scat_wait(0)
    scat_wait(1)

    plsc.subcore_barrier()
    pltpu.sync_copy(acc_sh.at[pl.ds(zb, RZ)], acc_out.at[c_id, pl.ds(zb, RZ)])


def _scatter_stage(gxv, xv1, eidx, cd16, zrows, nch):
    mesh = plsc.VectorSubcoreMesh(core_axis_name="c", subcore_axis_name="s")
    kfn = functools.partial(
        pl.kernel,
        out_type=jax.ShapeDtypeStruct((2, NP, W), jnp.float32),
        mesh=mesh,
        scratch_types=[
            pltpu.VMEM((3, 8 * 2 * CH), jnp.int32),
            pltpu.VMEM((CH,), jnp.int32),
            pltpu.VMEM((CH,), jnp.int32),
            pltpu.VMEM((CH, GW), jnp.float32),
            pltpu.VMEM((CH, GW), jnp.float32),
            pltpu.VMEM((CH, W), jnp.float32),
            pltpu.VMEM((CH, W), jnp.float32),
            pltpu.VMEM((NXV,), jnp.float32),
            pltpu.VMEM((16,), jnp.float32),
            pltpu.VMEM_SHARED((NP, W), jnp.float32),
            pltpu.SemaphoreType.DMA,
            pltpu.SemaphoreType.DMA,
            pltpu.SemaphoreType.DMA,
            pltpu.SemaphoreType.DMA,
            pltpu.SemaphoreType.DMA,
        ],
        compiler_params=pltpu.CompilerParams(
            needs_layout_passes=False, use_tc_tiling_on_sc=False),
    )(functools.partial(_sc_body, nch))
    return kfn(gxv, xv1, eidx, cd16, zrows)


def _fin_body(x_ref, gxv_ref, acc_ref, bias_ref, c_ref, o_ref):
    cd = c_ref[0, 0] - c_ref[0, 1]
    s0 = 1.0 / (1.0 + jnp.exp(-cd))
    self_msg = gxv_ref[:, D:2 * D] + s0 * gxv_ref[:, :D]
    summed = acc_ref[0, :, :D] + acc_ref[1, :, :D] + self_msg
    cnt = 1.0 + acc_ref[0, :, D:D + 1] + acc_ref[1, :, D:D + 1]
    conv = summed / cnt + bias_ref[0]
    o_ref[...] = x_ref[...] + jnp.maximum(conv, 0.0)


def _finalize(x, gxv, acc, bias, c2):
    R = 1024
    return pl.pallas_call(
        _fin_body,
        grid=(-(-N // R),),
        in_specs=[
            pl.BlockSpec((R, D), lambda i: (i, _z(i))),
            pl.BlockSpec((R, GW), lambda i: (i, _z(i))),
            pl.BlockSpec((2, R, W), lambda i: (_z(i), i, _z(i))),
            pl.BlockSpec((1, D), lambda i: (_z(i), _z(i))),
            pl.BlockSpec((1, 2), lambda i: (_z(i), _z(i))),
        ],
        out_specs=pl.BlockSpec((R, D), lambda i: (i, _z(i))),
        out_shape=jax.ShapeDtypeStruct((N, D), jnp.float32),
    )(x, gxv, acc, bias, c2)


def kernel(x, edge_index, weight, u, c, bias):
    E = edge_index.shape[1]
    src = edge_index[0].astype(jnp.int32)
    dst = edge_index[1].astype(jnp.int32)
    nch = -(-E // (NW * CH))
    if nch % 8:
        nch += 8 - nch % 8
    ept = nch * CH
    pad = ept * NW - E
    if pad:
        src = jnp.concatenate([src, jnp.zeros((pad,), jnp.int32)])
        dst = jnp.concatenate([dst, jnp.zeros((pad,), jnp.int32)])
    eidx = jnp.stack([src.reshape(-1, CH), dst.reshape(-1, CH)],
                     axis=1).reshape(-1)
    eidx = jnp.concatenate([eidx, jnp.zeros((2 * 8 * 2 * CH,), jnp.int32)])

    c2 = jnp.reshape(c, (1, 2)).astype(jnp.float32)
    gxv, xv1 = _prep(x, weight, u)
    xvp = jnp.pad(jnp.reshape(xv1, (N,)), (0, NXV - N))
    cd16 = jnp.broadcast_to(jnp.reshape(c[0] - c[1], (1,)), (16,)).astype(jnp.float32)
    zrows = jnp.zeros((NP, W), jnp.float32)
    acc = _scatter_stage(gxv, xvp, eidx, cd16, zrows, nch)
    return _finalize(x, gxv, acc,
                     jnp.reshape(bias, (1, D)).astype(jnp.float32), c2)


# speedup vs baseline: 1.2611x; 1.2611x over previous
import functools

import jax
import jax.numpy as jnp
from jax import lax
from jax.experimental import pallas as pl
from jax.experimental.pallas import tpu as pltpu
from jax.experimental.pallas import tpu_sc as plsc

N = 10000
D = 128
W = D + 16
GW = 2 * D
NXV = 10016
NP = 10016
CH = 16
NW = 32
RZ = NP // 16


def _z(i):
    return i * 0


def _prep_body(x_ref, w_ref, u_ref, gxv_ref, xv_ref):
    xw = jnp.dot(x_ref[...], w_ref[...],
                 preferred_element_type=jnp.float32,
                 precision=lax.Precision.HIGHEST)
    gxv_ref[:, :D] = xw[:, :D] - xw[:, D:]
    gxv_ref[:, D:2 * D] = xw[:, D:]
    uv = u_ref[:, 0:1] - u_ref[:, 1:2]
    xv = jnp.dot(x_ref[...], uv,
                 preferred_element_type=jnp.float32,
                 precision=lax.Precision.HIGHEST)
    xv_ref[...] = xv


def _prep(x, weight, u):
    R = 1000
    return pl.pallas_call(
        _prep_body,
        grid=(N // R,),
        in_specs=[
            pl.BlockSpec((R, D), lambda i: (i, _z(i))),
            pl.BlockSpec((D, 2 * D), lambda i: (_z(i), _z(i))),
            pl.BlockSpec((D, 2), lambda i: (_z(i), _z(i))),
        ],
        out_specs=[
            pl.BlockSpec((R, GW), lambda i: (i, _z(i))),
            pl.BlockSpec((R, 1), lambda i: (i, _z(i))),
        ],
        out_shape=[
            jax.ShapeDtypeStruct((N, GW), jnp.float32),
            jax.ShapeDtypeStruct((N, 1), jnp.float32),
        ],
    )(x, weight, u)


def _sc_body(nch, gxv_hbm, xv_hbm, eidx_hbm, cd_hbm, z_hbm,
             acc_out,
             idx0, idx1, idx2, idx3, dstm0, dstm1, dstm2, dstm3,
             rows0, rows1, rows2, rows3, msg0, msg1, msg2, msg3,
             xvl, cd_v, acc_sh,
             semi0, semi1, semi2, semi3, semg0, semg1, semg2, semg3,
             sems0, sems1, sems2, sems3):
    i32 = jnp.int32
    c_id = lax.axis_index("c")
    s_id = lax.axis_index("s")
    wid = c_id * i32(16) + s_id

    idx = (idx0, idx1, idx2, idx3)
    dstm = (dstm0, dstm1, dstm2, dstm3)
    rows = (rows0, rows1, rows2, rows3)
    msg = (msg0, msg1, msg2, msg3)
    semi = (semi0, semi1, semi2, semi3)
    semg = (semg0, semg1, semg2, semg3)
    sems = (sems0, sems1, sems2, sems3)

    zb = s_id * i32(RZ)
    pltpu.sync_copy(z_hbm.at[pl.ds(zb, RZ)], acc_sh.at[pl.ds(zb, RZ)])
    pltpu.sync_copy(xv_hbm, xvl)
    pltpu.sync_copy(cd_hbm, cd_v)
    ones16 = jnp.ones((16,), jnp.float32)
    for p in range(4):
        for r in range(CH):
            msg[p][i32(r), pl.ds(D, 16)] = ones16
    plsc.subcore_barrier()

    cbase = wid * i32(nch)

    def idx_copy(ci, p):
        return pltpu.make_async_copy(
            eidx_hbm.at[pl.ds((cbase + ci) * i32(2 * CH), 2 * CH)],
            idx[p], semi[p])

    def rows_copy(p):
        return pltpu.make_async_copy(
            gxv_hbm.at[idx[p].at[pl.ds(0, CH)]], rows[p], semg[p])

    def scat_start(p):
        pltpu.async_copy(msg[p], acc_sh.at[dstm[p]], sems[p], add=True)

    def scat_wait(p):
        pltpu.make_async_copy(msg[p], acc_sh.at[dstm[p]], sems[p]).wait()

    def compute(p):
        cdv = cd_v[...]
        srcv = idx[p][pl.ds(0, 16)]
        dstv = idx[p][pl.ds(CH, 16)]
        xvs = plsc.load_gather(xvl, [srcv])
        xvdv = plsc.load_gather(xvl, [dstv])
        q = 1.0 / (1.0 + jnp.exp(-(xvs - xvdv + cdv)))
        dstm[p][pl.ds(0, 16)] = jnp.where(srcv != dstv, dstv, i32(N))
        for e in range(16):
            qe = q[e]
            for k in range(D // 16):
                col = k * 16
                gseg = rows[p][e, pl.ds(col, 16)]
                bseg = rows[p][e, pl.ds(D + col, 16)]
                msg[p][e, pl.ds(col, 16)] = bseg + qe * gseg

    for j in range(4):
        idx_copy(i32(j), j).start()
    idx_copy(i32(0), 0).wait()
    rows_copy(0).start()
    idx_copy(i32(0), 1).wait()
    rows_copy(1).start()

    def quad(iq, carry):
        t0 = iq * i32(4)
        for j in range(4):
            t = t0 + i32(j)
            jg = (j + 2) % 4
            rows_copy(j).wait()

            @pl.when(t0 >= i32(4))
            def _(j=j):
                scat_wait(j)

            compute(j)
            idx_copy(t + i32(4), j).start()
            idx_copy(i32(0), jg).wait()
            rows_copy(jg).start()
            scat_start(j)
        return carry

    lax.fori_loop(jnp.int32(0), jnp.int32(nch // 4), quad, 0)

    idx_copy(i32(0), 2).wait()
    idx_copy(i32(0), 3).wait()
    rows_copy(0).wait()
    rows_copy(1).wait()
    for j in range(4):
        scat_wait(j)

    plsc.subcore_barrier()
    pltpu.sync_copy(acc_sh.at[pl.ds(zb, RZ)], acc_out.at[c_id, pl.ds(zb, RZ)])


def _scatter_stage(gxv, xv1, eidx, cd16, zrows, nch):
    mesh = plsc.VectorSubcoreMesh(core_axis_name="c", subcore_axis_name="s")
    kfn = functools.partial(
        pl.kernel,
        out_type=jax.ShapeDtypeStruct((2, NP, W), jnp.float32),
        mesh=mesh,
        scratch_types=(
            [pltpu.VMEM((2 * CH,), jnp.int32)] * 4
            + [pltpu.VMEM((CH,), jnp.int32)] * 4
            + [pltpu.VMEM((CH, GW), jnp.float32)] * 4
            + [pltpu.VMEM((CH, W), jnp.float32)] * 4
            + [pltpu.VMEM((NXV,), jnp.float32),
               pltpu.VMEM((16,), jnp.float32),
               pltpu.VMEM_SHARED((NP, W), jnp.float32)]
            + [pltpu.SemaphoreType.DMA] * 12
        ),
        compiler_params=pltpu.CompilerParams(
            needs_layout_passes=False, use_tc_tiling_on_sc=False),
    )(functools.partial(_sc_body, nch))
    return kfn(gxv, xv1, eidx, cd16, zrows)


def _fin_body(x_ref, gxv_ref, acc_ref, bias_ref, c_ref, o_ref):
    cd = c_ref[0, 0] - c_ref[0, 1]
    s0 = 1.0 / (1.0 + jnp.exp(-cd))
    self_msg = gxv_ref[:, D:2 * D] + s0 * gxv_ref[:, :D]
    summed = acc_ref[0, :, :D] + acc_ref[1, :, :D] + self_msg
    cnt = 1.0 + acc_ref[0, :, D:D + 1] + acc_ref[1, :, D:D + 1]
    conv = summed / cnt + bias_ref[0]
    o_ref[...] = x_ref[...] + jnp.maximum(conv, 0.0)


def _finalize(x, gxv, acc, bias, c2):
    R = 1024
    return pl.pallas_call(
        _fin_body,
        grid=(-(-N // R),),
        in_specs=[
            pl.BlockSpec((R, D), lambda i: (i, _z(i))),
            pl.BlockSpec((R, GW), lambda i: (i, _z(i))),
            pl.BlockSpec((2, R, W), lambda i: (_z(i), i, _z(i))),
            pl.BlockSpec((1, D), lambda i: (_z(i), _z(i))),
            pl.BlockSpec((1, 2), lambda i: (_z(i), _z(i))),
        ],
        out_specs=pl.BlockSpec((R, D), lambda i: (i, _z(i))),
        out_shape=jax.ShapeDtypeStruct((N, D), jnp.float32),
    )(x, gxv, acc, bias, c2)


def kernel(x, edge_index, weight, u, c, bias):
    E = edge_index.shape[1]
    src = edge_index[0].astype(jnp.int32)
    dst = edge_index[1].astype(jnp.int32)
    nch = -(-E // (NW * CH))
    if nch % 4:
        nch += 4 - nch % 4
    ept = nch * CH
    pad = ept * NW - E
    if pad:
        src = jnp.concatenate([src, jnp.zeros((pad,), jnp.int32)])
        dst = jnp.concatenate([dst, jnp.zeros((pad,), jnp.int32)])
    eidx = jnp.stack([src.reshape(-1, CH), dst.reshape(-1, CH)],
                     axis=1).reshape(-1)
    eidx = jnp.concatenate([eidx, jnp.zeros((8 * CH,), jnp.int32)])

    c2 = jnp.reshape(c, (1, 2)).astype(jnp.float32)
    gxv, xv1 = _prep(x, weight, u)
    xvp = jnp.pad(jnp.reshape(xv1, (N,)), (0, NXV - N))
    cd16 = jnp.broadcast_to(jnp.reshape(c[0] - c[1], (1,)), (16,)).astype(jnp.float32)
    zrows = jnp.zeros((NP, W), jnp.float32)
    acc = _scatter_stage(gxv, xvp, eidx, cd16, zrows, nch)
    return _finalize(x, gxv, acc,
                     jnp.reshape(bias, (1, D)).astype(jnp.float32), c2)


# speedup vs baseline: 1.4075x; 1.1161x over previous
import functools

import jax
import jax.numpy as jnp
from jax import lax
from jax.experimental import pallas as pl
from jax.experimental.pallas import tpu as pltpu
from jax.experimental.pallas import tpu_sc as plsc

N = 10000
D = 128
W = D + 16
GW = 2 * D
NXV = 10016
NP = 10112
CH = 32
NW = 32
RZ = NP // 16


def _z(i):
    return i * 0


def _prep_body(x_ref, w_ref, u_ref, gxv_ref, xv_ref):
    xw = jnp.dot(x_ref[...], w_ref[...],
                 preferred_element_type=jnp.float32,
                 precision=lax.Precision.HIGHEST)
    gxv_ref[:, :D] = xw[:, :D] - xw[:, D:]
    gxv_ref[:, D:2 * D] = xw[:, D:]
    uv = u_ref[:, 0:1] - u_ref[:, 1:2]
    xv = jnp.dot(x_ref[...], uv,
                 preferred_element_type=jnp.float32,
                 precision=lax.Precision.HIGHEST)
    xv_ref[...] = xv


def _prep(x, weight, u):
    R = 1000
    return pl.pallas_call(
        _prep_body,
        grid=(N // R,),
        in_specs=[
            pl.BlockSpec((R, D), lambda i: (i, _z(i))),
            pl.BlockSpec((D, 2 * D), lambda i: (_z(i), _z(i))),
            pl.BlockSpec((D, 2), lambda i: (_z(i), _z(i))),
        ],
        out_specs=[
            pl.BlockSpec((R, GW), lambda i: (i, _z(i))),
            pl.BlockSpec((R, 1), lambda i: (i, _z(i))),
        ],
        out_shape=[
            jax.ShapeDtypeStruct((N, GW), jnp.float32),
            jax.ShapeDtypeStruct((N, 1), jnp.float32),
        ],
    )(x, weight, u)


def _sc_body(nch, gxv_hbm, xv_hbm, eidx_hbm, cd_hbm, z_hbm,
             acc_out,
             idx0, idx1, dstm0, dstm1, rows0, rows1,
             msg0, msg1, xvl, cd_v, acc_sh,
             semi0, semi1, semg0, semg1, sems0, sems1):
    i32 = jnp.int32
    c_id = lax.axis_index("c")
    s_id = lax.axis_index("s")
    wid = c_id * i32(16) + s_id

    idx = (idx0, idx1)
    dstm = (dstm0, dstm1)
    rows = (rows0, rows1)
    msg = (msg0, msg1)
    semi = (semi0, semi1)
    semg = (semg0, semg1)
    sems = (sems0, sems1)

    zb = s_id * i32(RZ)
    pltpu.sync_copy(z_hbm.at[pl.ds(zb, RZ)], acc_sh.at[pl.ds(zb, RZ)])
    pltpu.sync_copy(xv_hbm, xvl)
    pltpu.sync_copy(cd_hbm, cd_v)
    ones16 = jnp.ones((16,), jnp.float32)
    for p in (0, 1):
        for r in range(CH):
            msg[p][i32(r), pl.ds(D, 16)] = ones16
    plsc.subcore_barrier()

    cbase = wid * i32(nch)

    def idx_copy(ci, p):
        return pltpu.make_async_copy(
            eidx_hbm.at[pl.ds((cbase + ci) * i32(2 * CH), 2 * CH)],
            idx[p], semi[p])

    def rows_copy(p):
        return pltpu.make_async_copy(
            gxv_hbm.at[idx[p].at[pl.ds(0, CH)]], rows[p], semg[p])

    def scat_start(p):
        pltpu.async_copy(msg[p], acc_sh.at[dstm[p]], sems[p], add=True)

    def scat_wait(p):
        pltpu.make_async_copy(msg[p], acc_sh.at[dstm[p]], sems[p]).wait()

    def start_gathers(p):
        rows_copy(p).start()

    def wait_gathers(p):
        rows_copy(p).wait()

    def compute(p):
        cdv = cd_v[...]
        for g in range(CH // 16):
            srcv = idx[p][pl.ds(g * 16, 16)]
            dstv = idx[p][pl.ds(CH + g * 16, 16)]
            xvs = plsc.load_gather(xvl, [srcv])
            xvdv = plsc.load_gather(xvl, [dstv])
            q = 1.0 / (1.0 + jnp.exp(-(xvs - xvdv + cdv)))
            dstm[p][pl.ds(g * 16, 16)] = jnp.where(srcv != dstv, dstv, i32(N))
            for e in range(16):
                qe = q[e]
                r = g * 16 + e
                for k in range(D // 16):
                    col = k * 16
                    gseg = rows[p][r, pl.ds(col, 16)]
                    bseg = rows[p][r, pl.ds(D + col, 16)]
                    msg[p][r, pl.ds(col, 16)] = bseg + qe * gseg

    idx_copy(i32(0), 0).start()
    idx_copy(i32(0), 0).wait()
    start_gathers(0)
    idx_copy(i32(1), 1).start()

    def pair(i2, carry):
        a = i2 * i32(2)
        wait_gathers(0)

        @pl.when(a >= i32(2))
        def _():
            scat_wait(0)

        idx_copy(i32(0), 1).wait()
        start_gathers(1)
        compute(0)
        idx_copy(a + i32(2), 0).start()
        scat_start(0)

        wait_gathers(1)

        @pl.when(a >= i32(1))
        def _():
            scat_wait(1)

        idx_copy(i32(0), 0).wait()
        start_gathers(0)
        compute(1)
        idx_copy(a + i32(3), 1).start()
        scat_start(1)
        return carry

    lax.fori_loop(jnp.int32(0), jnp.int32(nch // 2), pair, 0)

    idx_copy(i32(0), 1).wait()
    wait_gathers(0)
    scat_wait(0)
    scat_wait(1)

    plsc.subcore_barrier()
    pltpu.sync_copy(acc_sh.at[pl.ds(zb, RZ)], acc_out.at[c_id, pl.ds(zb, RZ)])


def _scatter_stage(gxv, xv1, eidx, cd16, zrows, nch):
    mesh = plsc.VectorSubcoreMesh(core_axis_name="c", subcore_axis_name="s")
    kfn = functools.partial(
        pl.kernel,
        out_type=jax.ShapeDtypeStruct((2, NP, W), jnp.float32),
        mesh=mesh,
        scratch_types=[
            pltpu.VMEM((2 * CH,), jnp.int32),
            pltpu.VMEM((2 * CH,), jnp.int32),
            pltpu.VMEM((CH,), jnp.int32),
            pltpu.VMEM((CH,), jnp.int32),
            pltpu.VMEM((CH, GW), jnp.float32),
            pltpu.VMEM((CH, GW), jnp.float32),
            pltpu.VMEM((CH, W), jnp.float32),
            pltpu.VMEM((CH, W), jnp.float32),
            pltpu.VMEM((NXV,), jnp.float32),
            pltpu.VMEM((16,), jnp.float32),
            pltpu.VMEM_SHARED((NP, W), jnp.float32),
            pltpu.SemaphoreType.DMA,
            pltpu.SemaphoreType.DMA,
            pltpu.SemaphoreType.DMA,
            pltpu.SemaphoreType.DMA,
            pltpu.SemaphoreType.DMA,
            pltpu.SemaphoreType.DMA,
        ],
        compiler_params=pltpu.CompilerParams(
            needs_layout_passes=False, use_tc_tiling_on_sc=False),
    )(functools.partial(_sc_body, nch))
    return kfn(gxv, xv1, eidx, cd16, zrows)


def _fin_body(x_ref, gxv_ref, acc_ref, bias_ref, c_ref, o_ref):
    cd = c_ref[0, 0] - c_ref[0, 1]
    s0 = 1.0 / (1.0 + jnp.exp(-cd))
    self_msg = gxv_ref[:, D:2 * D] + s0 * gxv_ref[:, :D]
    summed = acc_ref[0, :, :D] + acc_ref[1, :, :D] + self_msg
    cnt = 1.0 + acc_ref[0, :, D:D + 1] + acc_ref[1, :, D:D + 1]
    conv = summed / cnt + bias_ref[0]
    o_ref[...] = x_ref[...] + jnp.maximum(conv, 0.0)


def _finalize(x, gxv, acc, bias, c2):
    R = 1024
    return pl.pallas_call(
        _fin_body,
        grid=(-(-N // R),),
        in_specs=[
            pl.BlockSpec((R, D), lambda i: (i, _z(i))),
            pl.BlockSpec((R, GW), lambda i: (i, _z(i))),
            pl.BlockSpec((2, R, W), lambda i: (_z(i), i, _z(i))),
            pl.BlockSpec((1, D), lambda i: (_z(i), _z(i))),
            pl.BlockSpec((1, 2), lambda i: (_z(i), _z(i))),
        ],
        out_specs=pl.BlockSpec((R, D), lambda i: (i, _z(i))),
        out_shape=jax.ShapeDtypeStruct((N, D), jnp.float32),
    )(x, gxv, acc, bias, c2)


def kernel(x, edge_index, weight, u, c, bias):
    E = edge_index.shape[1]
    src = edge_index[0].astype(jnp.int32)
    dst = edge_index[1].astype(jnp.int32)
    nch = -(-E // (NW * CH))
    if nch % 2:
        nch += 1
    ept = nch * CH
    pad = ept * NW - E
    if pad:
        src = jnp.concatenate([src, jnp.zeros((pad,), jnp.int32)])
        dst = jnp.concatenate([dst, jnp.zeros((pad,), jnp.int32)])
    eidx = jnp.stack([src.reshape(-1, CH), dst.reshape(-1, CH)],
                     axis=1).reshape(-1)
    eidx = jnp.concatenate([eidx, jnp.zeros((4 * CH,), jnp.int32)])

    c2 = jnp.reshape(c, (1, 2)).astype(jnp.float32)
    gxv, xv1 = _prep(x, weight, u)
    xvp = jnp.pad(jnp.reshape(xv1, (N,)), (0, NXV - N))
    cd16 = jnp.broadcast_to(jnp.reshape(c[0] - c[1], (1,)), (16,)).astype(jnp.float32)
    zrows = jnp.zeros((NP, W), jnp.float32)
    acc = _scatter_stage(gxv, xvp, eidx, cd16, zrows, nch)
    return _finalize(x, gxv, acc,
                     jnp.reshape(bias, (1, D)).astype(jnp.float32), c2)
